# baseline (speedup 1.0000x reference)
import functools

import numpy as np
import jax
import jax.numpy as jnp
from jax import lax
from jax.experimental import pallas as pl
from jax.experimental.pallas import tpu as pltpu
from jax.experimental.pallas import tpu_sc as plsc

_B, _H, _W = 64, 50, 50
_P = _H * _W
_N = _B * _P
_UD = 32


_NW = 32
_BPW = _N // _NW
_CHUNK = 1000


def _sc_gather_unit(table, idx_flat):
    mesh = plsc.VectorSubcoreMesh(core_axis_name="c", subcore_axis_name="s")

    @functools.partial(
        pl.kernel,
        mesh=mesh,
        out_type=jax.ShapeDtypeStruct((_N, _UD), jnp.float32),
        scratch_types=[
            pltpu.VMEM((_CHUNK,), jnp.int32),
            pltpu.VMEM((_CHUNK, _UD), jnp.float32),
            pltpu.SemaphoreType.DMA,
        ],
        compiler_params=pltpu.CompilerParams(use_tc_tiling_on_sc=False),
    )
    def k(table_hbm, idx_hbm, out_hbm, idx_v, rows_v, sem):
        wid = lax.axis_index("s") * 2 + lax.axis_index("c")
        base = wid * _BPW

        @pl.loop(0, _BPW, step=_CHUNK)
        def _(off):
            pltpu.sync_copy(idx_hbm.at[pl.ds(base + off, _CHUNK)], idx_v)
            pltpu.async_copy(table_hbm.at[idx_v], rows_v, sem).wait()
            pltpu.sync_copy(rows_v, out_hbm.at[pl.ds(base + off, _CHUNK)])

    return k(table, idx_flat)


_SIZES = [14, 14, 14, 14, 14, 14, 12, 12, 12, 12, 4, 4]
_L = sum(_SIZES)
_COLID = np.repeat(np.arange(12), _SIZES)
_VAL = np.concatenate([np.arange(s) for s in _SIZES])
_SCORE_OF_COL = np.array([-1, -1, 0, 1, 2, 3, 4, 5, 6, 7, 8, 9])
_SCORE = _SCORE_OF_COL[_COLID]
_GROUP_OF_COL = np.array([0, 0, 1, 1, 1, 1, 2, 2, 2, 2, 3, 3])
_GRP = _GROUP_OF_COL[_COLID]

_S_NP = (np.arange(12)[:, None] == _COLID[None, :]).astype(np.float32)
_VAL_NP = _VAL[None, :].astype(np.float32)
_SM_MASK_NP = (_SCORE[:, None] == np.arange(10)[None, :]).astype(np.float32)
_E_NP = (_SCORE[None, :] == np.arange(10)[:, None]).astype(np.float32)
_SETG = np.array([0, 0, 0, 0, 1, 1, 1, 1, 2, 2])
_G_NP = (_SETG[:, None] == np.arange(3)[None, :]).astype(np.float32)
_GT_NP = _G_NP.T.copy()
_THALF_NP = np.where(_GRP == 0, 0.5, 0.0)[None, :].astype(np.float32)
_COLSEL_NP = (_GRP[:, None] == np.arange(4)[None, :]).astype(np.float32)


_G = 4
_F = 65


def _assemble_body(feats_ref, uemb_ref, s_ref, val_ref, sm_ref, e_ref,
                   g_ref, gt_ref, thalf_ref, tall_ref, out_ref):
    f32 = jnp.float32
    r = lax.broadcasted_iota(jnp.int32, (_P, 1), 0)
    px = (r % _W).astype(f32) * (1.0 / _W)
    py = (r // _W).astype(f32) * (1.0 / _H)

    for g in range(_G):
        f = feats_ref[g]
        idxf = f[:, 0:12]
        mask = f[:, 12:13]

        idx_wide = jnp.dot(idxf, s_ref[...], preferred_element_type=f32)
        oh = (jnp.abs(idx_wide - val_ref[...]) < 0.5).astype(f32)

        scores = jnp.dot(oh, sm_ref[...], preferred_element_type=f32)
        e = jnp.exp(scores)
        gs = jnp.dot(e, g_ref[...], preferred_element_type=f32)
        winv = jnp.dot(1.0 / gs, gt_ref[...], preferred_element_type=f32)
        w = e * winv * mask
        wex = (jnp.dot(w, e_ref[...], preferred_element_type=f32)
               + thalf_ref[...])
        dense = jnp.dot(oh * wex, tall_ref[...], preferred_element_type=f32)

        out_ref[g] = jnp.concatenate([
            dense[:, 0:16], px, py,
            uemb_ref[g] * mask, f[:, 13:24] * mask,
            dense[:, 16:64],
            f[:, 24:30] * mask, f[:, 30:46] * mask,
            f[:, 46:62] * (mask * 0.1),
            f[:, 62:65],
        ], axis=1)


def _assemble(feats, uemb,
              s_m, val_m, sm_m, e_m, g_m, gt_m, thalf_m, tall_m,
              interpret=False):
    def img_spec(k):
        return pl.BlockSpec((_G, _P, k), lambda b: (b, 0, 0))

    def full_spec(shape):
        return pl.BlockSpec(shape, lambda b: (0, 0))

    return pl.pallas_call(
        _assemble_body,
        grid=(_B // _G,),
        in_specs=[
            img_spec(_F), img_spec(_UD),
            full_spec((12, _L)), full_spec((1, _L)), full_spec((_L, 10)),
            full_spec((10, _L)), full_spec((10, 3)), full_spec((3, 10)),
            full_spec((1, _L)), full_spec((_L, 64)),
        ],
        out_specs=pl.BlockSpec((_G, _P, 150), lambda b: (b, 0, 0)),
        out_shape=jax.ShapeDtypeStruct((_B, _P, 150), jnp.float32),
        compiler_params=pltpu.CompilerParams(
            dimension_semantics=("parallel",)),
        interpret=interpret,
    )(feats, uemb, s_m, val_m, sm_m, e_m, g_m, gt_m, thalf_m, tall_m)


def _prep_constants(terrain_table, ability_table, trait_table, status_table,
                    ability_query, trait_query, status_query):
    f32 = jnp.float32
    sv_a = jnp.einsum("rd,d->r", ability_table, ability_query)
    sv_r = jnp.einsum("rd,d->r", trait_table, trait_query)
    sv_s = jnp.einsum("rd,d->r", status_table, status_query)
    sv_cat = jnp.concatenate([
        jnp.zeros((28,), f32),
        jnp.tile(sv_a, 4), jnp.tile(sv_r, 4), jnp.tile(sv_s, 2),
    ])
    sm_m = sv_cat[:, None] * jnp.asarray(_SM_MASK_NP)

    r_stack = jnp.concatenate(
        [terrain_table] * 2 + [ability_table] * 4 + [trait_table] * 4
        + [status_table] * 2, axis=0)
    tall_m = (r_stack[:, None, :]
              * jnp.asarray(_COLSEL_NP)[:, :, None]).reshape(_L, 64)
    return sm_m, tall_m


def kernel(terrain_idx, unit_type_idx, ability_idx, trait_idx, status_idx,
           unit_mask, numerical, resistances, defenses, movement_costs,
           modifier_flags, terrain_table, unit_type_table, ability_table,
           trait_table, status_table, ability_query, trait_query,
           status_query):
    uemb = _sc_gather_unit(unit_type_table, unit_type_idx.reshape(_N))

    f32 = jnp.float32
    feats = jnp.concatenate([
        terrain_idx.reshape(_B, _P, 2).astype(f32),
        ability_idx.reshape(_B, _P, 4).astype(f32),
        trait_idx.reshape(_B, _P, 4).astype(f32),
        status_idx.reshape(_B, _P, 2).astype(f32),
        unit_mask.reshape(_B, _P, 1),
        numerical.reshape(_B, _P, 11),
        resistances.reshape(_B, _P, 6),
        defenses.reshape(_B, _P, 16),
        movement_costs.reshape(_B, _P, 16),
        modifier_flags.reshape(_B, _P, 3),
    ], axis=-1)

    sm_m, tall_m = _prep_constants(
        terrain_table, ability_table, trait_table, status_table,
        ability_query, trait_query, status_query)

    out = _assemble(
        feats,
        uemb.reshape(_B, _P, _UD),
        jnp.asarray(_S_NP), jnp.asarray(_VAL_NP), sm_m,
        jnp.asarray(_E_NP), jnp.asarray(_G_NP), jnp.asarray(_GT_NP),
        jnp.asarray(_THALF_NP), tall_m,
    )
    return out.reshape(_B, _H, _W, 150)


# speedup vs baseline: 1.0072x; 1.0072x over previous
import functools

import numpy as np
import jax
import jax.numpy as jnp
from jax import lax
from jax.experimental import pallas as pl
from jax.experimental.pallas import tpu as pltpu
from jax.experimental.pallas import tpu_sc as plsc

_B, _H, _W = 64, 50, 50
_P = _H * _W
_N = _B * _P
_UD = 32


_NW = 32
_BPW = _N // _NW
_CHUNK = 1000


def _sc_gather_unit(table, idx_flat):
    mesh = plsc.VectorSubcoreMesh(core_axis_name="c", subcore_axis_name="s")

    @functools.partial(
        pl.kernel,
        mesh=mesh,
        out_type=jax.ShapeDtypeStruct((_N, _UD), jnp.float32),
        scratch_types=[
            pltpu.VMEM((_CHUNK,), jnp.int32),
            pltpu.VMEM((_CHUNK, _UD), jnp.float32),
            pltpu.SemaphoreType.DMA,
        ],
        compiler_params=pltpu.CompilerParams(use_tc_tiling_on_sc=False),
    )
    def k(table_hbm, idx_hbm, out_hbm, idx_v, rows_v, sem):
        wid = lax.axis_index("s") * 2 + lax.axis_index("c")
        base = wid * _BPW

        @pl.loop(0, _BPW, step=_CHUNK)
        def _(off):
            pltpu.sync_copy(idx_hbm.at[pl.ds(base + off, _CHUNK)], idx_v)
            pltpu.async_copy(table_hbm.at[idx_v], rows_v, sem).wait()
            pltpu.sync_copy(rows_v, out_hbm.at[pl.ds(base + off, _CHUNK)])

    return k(table, idx_flat)


_SIZES = [14, 14, 14, 14, 14, 14, 12, 12, 12, 12, 4, 4]
_L = sum(_SIZES)
_COLID = np.repeat(np.arange(12), _SIZES)
_VAL = np.concatenate([np.arange(s) for s in _SIZES])
_SCORE_OF_COL = np.array([-1, -1, 0, 1, 2, 3, 4, 5, 6, 7, 8, 9])
_SCORE = _SCORE_OF_COL[_COLID]
_GROUP_OF_COL = np.array([0, 0, 1, 1, 1, 1, 2, 2, 2, 2, 3, 3])
_GRP = _GROUP_OF_COL[_COLID]

_S_NP = (np.arange(12)[:, None] == _COLID[None, :]).astype(np.float32)
_VAL_NP = _VAL[None, :].astype(np.float32)
_SM_MASK_NP = (_SCORE[:, None] == np.arange(10)[None, :]).astype(np.float32)
_E_NP = (_SCORE[None, :] == np.arange(10)[:, None]).astype(np.float32)
_SETG = np.array([0, 0, 0, 0, 1, 1, 1, 1, 2, 2])
_G_NP = (_SETG[:, None] == np.arange(3)[None, :]).astype(np.float32)
_GT_NP = _G_NP.T.copy()
_THALF_NP = np.where(_GRP == 0, 0.5, 0.0)[None, :].astype(np.float32)
_COLSEL_NP = (_GRP[:, None] == np.arange(4)[None, :]).astype(np.float32)


_G = 1
_F = 65


def _assemble_body(feats_ref, uemb_ref, s_ref, val_ref, sm_ref, e_ref,
                   g_ref, gt_ref, thalf_ref, tall_ref, out_ref):
    f32 = jnp.float32
    r = lax.broadcasted_iota(jnp.int32, (_P, 1), 0)
    px = (r % _W).astype(f32) * (1.0 / _W)
    py = (r // _W).astype(f32) * (1.0 / _H)

    for g in range(_G):
        f = feats_ref[g]
        idxf = f[:, 0:12]
        mask = f[:, 12:13]

        idx_wide = jnp.dot(idxf, s_ref[...], preferred_element_type=f32)
        oh = (jnp.abs(idx_wide - val_ref[...]) < 0.5).astype(f32)

        scores = jnp.dot(oh, sm_ref[...], preferred_element_type=f32)
        e = jnp.exp(scores)
        gs = jnp.dot(e, g_ref[...], preferred_element_type=f32)
        winv = jnp.dot(1.0 / gs, gt_ref[...], preferred_element_type=f32)
        w = e * winv * mask
        wex = (jnp.dot(w, e_ref[...], preferred_element_type=f32)
               + thalf_ref[...])
        dense = jnp.dot(oh * wex, tall_ref[...], preferred_element_type=f32)

        out_ref[g] = jnp.concatenate([
            dense[:, 0:16], px, py,
            uemb_ref[g] * mask, f[:, 13:24] * mask,
            dense[:, 16:64],
            f[:, 24:30] * mask, f[:, 30:46] * mask,
            f[:, 46:62] * (mask * 0.1),
            f[:, 62:65],
        ], axis=1)


def _assemble(feats, uemb,
              s_m, val_m, sm_m, e_m, g_m, gt_m, thalf_m, tall_m,
              interpret=False):
    def img_spec(k):
        return pl.BlockSpec((_G, _P, k), lambda b: (b, 0, 0))

    def full_spec(shape):
        return pl.BlockSpec(shape, lambda b: (0, 0))

    return pl.pallas_call(
        _assemble_body,
        grid=(_B // _G,),
        in_specs=[
            img_spec(_F), img_spec(_UD),
            full_spec((12, _L)), full_spec((1, _L)), full_spec((_L, 10)),
            full_spec((10, _L)), full_spec((10, 3)), full_spec((3, 10)),
            full_spec((1, _L)), full_spec((_L, 64)),
        ],
        out_specs=pl.BlockSpec((_G, _P, 150), lambda b: (b, 0, 0)),
        out_shape=jax.ShapeDtypeStruct((_B, _P, 150), jnp.float32),
        compiler_params=pltpu.CompilerParams(
            dimension_semantics=("parallel",)),
        interpret=interpret,
    )(feats, uemb, s_m, val_m, sm_m, e_m, g_m, gt_m, thalf_m, tall_m)


def _prep_constants(terrain_table, ability_table, trait_table, status_table,
                    ability_query, trait_query, status_query):
    f32 = jnp.float32
    sv_a = jnp.einsum("rd,d->r", ability_table, ability_query)
    sv_r = jnp.einsum("rd,d->r", trait_table, trait_query)
    sv_s = jnp.einsum("rd,d->r", status_table, status_query)
    sv_cat = jnp.concatenate([
        jnp.zeros((28,), f32),
        jnp.tile(sv_a, 4), jnp.tile(sv_r, 4), jnp.tile(sv_s, 2),
    ])
    sm_m = sv_cat[:, None] * jnp.asarray(_SM_MASK_NP)

    r_stack = jnp.concatenate(
        [terrain_table] * 2 + [ability_table] * 4 + [trait_table] * 4
        + [status_table] * 2, axis=0)
    tall_m = (r_stack[:, None, :]
              * jnp.asarray(_COLSEL_NP)[:, :, None]).reshape(_L, 64)
    return sm_m, tall_m


def kernel(terrain_idx, unit_type_idx, ability_idx, trait_idx, status_idx,
           unit_mask, numerical, resistances, defenses, movement_costs,
           modifier_flags, terrain_table, unit_type_table, ability_table,
           trait_table, status_table, ability_query, trait_query,
           status_query):
    uemb = _sc_gather_unit(unit_type_table, unit_type_idx.reshape(_N))

    f32 = jnp.float32
    feats = jnp.concatenate([
        terrain_idx.reshape(_B, _P, 2).astype(f32),
        ability_idx.reshape(_B, _P, 4).astype(f32),
        trait_idx.reshape(_B, _P, 4).astype(f32),
        status_idx.reshape(_B, _P, 2).astype(f32),
        unit_mask.reshape(_B, _P, 1),
        numerical.reshape(_B, _P, 11),
        resistances.reshape(_B, _P, 6),
        defenses.reshape(_B, _P, 16),
        movement_costs.reshape(_B, _P, 16),
        modifier_flags.reshape(_B, _P, 3),
    ], axis=-1)

    sm_m, tall_m = _prep_constants(
        terrain_table, ability_table, trait_table, status_table,
        ability_query, trait_query, status_query)

    out = _assemble(
        feats,
        uemb.reshape(_B, _P, _UD),
        jnp.asarray(_S_NP), jnp.asarray(_VAL_NP), sm_m,
        jnp.asarray(_E_NP), jnp.asarray(_G_NP), jnp.asarray(_GT_NP),
        jnp.asarray(_THALF_NP), tall_m,
    )
    return out.reshape(_B, _H, _W, 150)


# speedup vs baseline: 2.3889x; 2.3718x over previous
import functools

import numpy as np
import jax
import jax.numpy as jnp
from jax import lax
from jax.experimental import pallas as pl
from jax.experimental.pallas import tpu as pltpu
from jax.experimental.pallas import tpu_sc as plsc

_B, _H, _W = 64, 50, 50
_P = _H * _W
_N = _B * _P
_UD = 32
_PR = _W * _B


_NW = 32
_BPW = _N // _NW
_CHUNK = 1000


def _sc_gather_unit(table, idx_flat):
    mesh = plsc.VectorSubcoreMesh(core_axis_name="c", subcore_axis_name="s")

    @functools.partial(
        pl.kernel,
        mesh=mesh,
        out_type=jax.ShapeDtypeStruct((_N, _UD), jnp.float32),
        scratch_types=[
            pltpu.VMEM((_CHUNK,), jnp.int32),
            pltpu.VMEM((_CHUNK, _UD), jnp.float32),
            pltpu.SemaphoreType.DMA,
        ],
        compiler_params=pltpu.CompilerParams(use_tc_tiling_on_sc=False),
    )
    def k(table_hbm, idx_hbm, out_hbm, idx_v, rows_v, sem):
        wid = lax.axis_index("s") * 2 + lax.axis_index("c")
        base = wid * _BPW

        @pl.loop(0, _BPW, step=_CHUNK)
        def _(off):
            pltpu.sync_copy(idx_hbm.at[pl.ds(base + off, _CHUNK)], idx_v)
            pltpu.async_copy(table_hbm.at[idx_v], rows_v, sem).wait()
            pltpu.sync_copy(rows_v, out_hbm.at[pl.ds(base + off, _CHUNK)])

    return k(table, idx_flat)


_SIZES = [14, 14, 14, 14, 14, 14, 12, 12, 12, 12, 4, 4]
_L = sum(_SIZES)
_COLID = np.repeat(np.arange(12), _SIZES)
_VAL = np.concatenate([np.arange(s) for s in _SIZES])
_SCORE_OF_COL = np.array([-1, -1, 0, 1, 2, 3, 4, 5, 6, 7, 8, 9])
_SCORE = _SCORE_OF_COL[_COLID]
_GROUP_OF_COL = np.array([0, 0, 1, 1, 1, 1, 2, 2, 2, 2, 3, 3])
_GRP = _GROUP_OF_COL[_COLID]

_S_NP = (np.arange(12)[:, None] == _COLID[None, :]).astype(np.float32)
_VAL_NP = _VAL[None, :].astype(np.float32)
_SM_MASK_NP = (_SCORE[:, None] == np.arange(10)[None, :]).astype(np.float32)
_E_NP = (_SCORE[None, :] == np.arange(10)[:, None]).astype(np.float32)
_SETG = np.array([0, 0, 0, 0, 1, 1, 1, 1, 2, 2])
_G_NP = (_SETG[:, None] == np.arange(3)[None, :]).astype(np.float32)
_GT_NP = _G_NP.T.copy()
_THALF_NP = np.where(_GRP == 0, 0.5, 0.0)[None, :].astype(np.float32)
_COLSEL_NP = (_GRP[:, None] == np.arange(4)[None, :]).astype(np.float32)


def _assemble_body(t_ref, a_ref, r_ref, st_ref, mask_ref, num_ref, res_ref,
                   def_ref, mov_ref, mod_ref, uemb_ref, s_ref, val_ref,
                   sm_ref, e_ref, g_ref, gt_ref, thalf_ref, tall_ref,
                   out_ref):
    f32 = jnp.float32

    def tr(x):
        return jnp.swapaxes(x, 1, 2).reshape(_PR, x.shape[1])

    tf = tr(t_ref[0])
    af = tr(a_ref[0])
    rf = tr(r_ref[0])
    stf = tr(st_ref[0])
    mask = tr(mask_ref[0][:, None, :])

    s = s_ref[...]
    idx_wide = (
        jnp.dot(tf, s[0:2], preferred_element_type=f32)
        + jnp.dot(af, s[2:6], preferred_element_type=f32)
        + jnp.dot(rf, s[6:10], preferred_element_type=f32)
        + jnp.dot(stf, s[10:12], preferred_element_type=f32))
    oh = (jnp.abs(idx_wide - val_ref[...]) < 0.5).astype(f32)

    scores = jnp.dot(oh, sm_ref[...], preferred_element_type=f32)
    e = jnp.exp(scores)
    gs = jnp.dot(e, g_ref[...], preferred_element_type=f32)
    winv = jnp.dot(1.0 / gs, gt_ref[...], preferred_element_type=f32)
    w = e * winv * mask
    wex = (jnp.dot(w, e_ref[...], preferred_element_type=f32)
           + thalf_ref[...])
    dense = jnp.dot(oh * wex, tall_ref[...], preferred_element_type=f32)

    h = pl.program_id(0)
    r = lax.broadcasted_iota(jnp.int32, (_PR, 1), 0)
    px = (r // _B).astype(f32) * (1.0 / _W)
    py = jnp.full((_PR, 1), 0.0, f32) + h.astype(f32) * (1.0 / _H)

    res = jnp.concatenate([
        dense[:, 0:16], px, py,
        uemb_ref[...] * mask, tr(num_ref[0]) * mask,
        dense[:, 16:64],
        tr(res_ref[0]) * mask, tr(def_ref[0]) * mask,
        tr(mov_ref[0]) * (mask * 0.1),
        tr(mod_ref[0]),
    ], axis=1)
    out_ref[0] = res.reshape(_W, _B, 150)


def _assemble(tf, af, rf, stf, mask_t, num_t, res_t, def_t, mov_t, mod_t,
              uemb, s_m, val_m, sm_m, e_m, g_m, gt_m, thalf_m, tall_m,
              interpret=False):
    def row_spec(k):
        return pl.BlockSpec((1, _W, k, _B), lambda b: (b, 0, 0, 0))

    def full_spec(shape):
        return pl.BlockSpec(shape, lambda b: (0, 0))

    return pl.pallas_call(
        _assemble_body,
        grid=(_H,),
        in_specs=[
            row_spec(2), row_spec(4), row_spec(4), row_spec(2),
            pl.BlockSpec((1, _W, _B), lambda b: (b, 0, 0)),
            row_spec(11), row_spec(6), row_spec(16), row_spec(16),
            row_spec(3),
            pl.BlockSpec((_PR, _UD), lambda b: (b, 0)),
            full_spec((12, _L)), full_spec((1, _L)), full_spec((_L, 10)),
            full_spec((10, _L)), full_spec((10, 3)), full_spec((3, 10)),
            full_spec((1, _L)), full_spec((_L, 64)),
        ],
        out_specs=pl.BlockSpec((1, _W, _B, 150), lambda b: (b, 0, 0, 0)),
        out_shape=jax.ShapeDtypeStruct((_H, _W, _B, 150), jnp.float32),
        compiler_params=pltpu.CompilerParams(
            dimension_semantics=("parallel",)),
        interpret=interpret,
    )(tf, af, rf, stf, mask_t, num_t, res_t, def_t, mov_t, mod_t, uemb,
      s_m, val_m, sm_m, e_m, g_m, gt_m, thalf_m, tall_m)


def _prep_constants(terrain_table, ability_table, trait_table, status_table,
                    ability_query, trait_query, status_query):
    f32 = jnp.float32
    sv_a = jnp.einsum("rd,d->r", ability_table, ability_query)
    sv_r = jnp.einsum("rd,d->r", trait_table, trait_query)
    sv_s = jnp.einsum("rd,d->r", status_table, status_query)
    sv_cat = jnp.concatenate([
        jnp.zeros((28,), f32),
        jnp.tile(sv_a, 4), jnp.tile(sv_r, 4), jnp.tile(sv_s, 2),
    ])
    sm_m = sv_cat[:, None] * jnp.asarray(_SM_MASK_NP)

    r_stack = jnp.concatenate(
        [terrain_table] * 2 + [ability_table] * 4 + [trait_table] * 4
        + [status_table] * 2, axis=0)
    tall_m = (r_stack[:, None, :]
              * jnp.asarray(_COLSEL_NP)[:, :, None]).reshape(_L, 64)
    return sm_m, tall_m


def kernel(terrain_idx, unit_type_idx, ability_idx, trait_idx, status_idx,
           unit_mask, numerical, resistances, defenses, movement_costs,
           modifier_flags, terrain_table, unit_type_table, ability_table,
           trait_table, status_table, ability_query, trait_query,
           status_query):
    f32 = jnp.float32

    idx_hex_major = jnp.transpose(unit_type_idx, (1, 2, 0)).reshape(_N)
    uemb = _sc_gather_unit(unit_type_table, idx_hex_major)

    def tview(x):
        return jnp.transpose(x, (1, 2, 3, 0))

    sm_m, tall_m = _prep_constants(
        terrain_table, ability_table, trait_table, status_table,
        ability_query, trait_query, status_query)

    out = _assemble(
        tview(terrain_idx).astype(f32),
        tview(ability_idx).astype(f32),
        tview(trait_idx).astype(f32),
        tview(status_idx).astype(f32),
        jnp.transpose(unit_mask, (1, 2, 0)),
        tview(numerical), tview(resistances), tview(defenses),
        tview(movement_costs), tview(modifier_flags),
        uemb,
        jnp.asarray(_S_NP), jnp.asarray(_VAL_NP), sm_m,
        jnp.asarray(_E_NP), jnp.asarray(_G_NP), jnp.asarray(_GT_NP),
        jnp.asarray(_THALF_NP), tall_m,
    )
    return jnp.transpose(out, (2, 0, 1, 3))
